# baseline (speedup 1.0000x reference)
import functools

import jax
import jax.numpy as jnp
from jax import lax
from jax.experimental import pallas as pl
from jax.experimental.pallas import tpu as pltpu
from jax.experimental.pallas import tpu_sc as plsc

_NC = 2
_NS = 16
_NW = _NC * _NS
_CH = 128
_LG = 13
_LB = 1 << _LG


def _bf16_bits(y):
    u = lax.bitcast_convert_type(y, jnp.int32)
    t = u + 32767 + (lax.shift_right_logical(u, 16) & 1)
    return lax.shift_right_logical(t, 16)


def _pack_body(x0_ref, x1_ref, x2_ref, x3_ref, eye_ref, o_ref):
    eye = eye_ref[...]
    dn = (((0,), (0,)), ((), ()))
    for k, ref in enumerate((x0_ref, x1_ref, x2_ref, x3_ref)):
        y = lax.dot_general(ref[...], eye, dn,
                            preferred_element_type=jnp.float32)
        b = _bf16_bits(y)
        o_ref[:, 32 * k:32 * k + 32] = b[:, :32] | lax.shift_left(b[:, 32:],
                                                                  16)


def _pack(table_t, n_rows, eye):
    nvalid = (n_rows + _LB - 1) // _LB
    grid = (nvalid + 3) // 4
    last = nvalid - 1
    mk = lambda k: pl.BlockSpec(
        (64, _LB), lambda i, k=k: (0, jnp.minimum(4 * i + k, last)))
    return pl.pallas_call(
        _pack_body,
        grid=(grid,),
        in_specs=[mk(0), mk(1), mk(2), mk(3),
                  pl.BlockSpec((64, 64), lambda i: (0, 0))],
        out_specs=pl.BlockSpec((_LB, 128), lambda i: (i, 0)),
        out_shape=jax.ShapeDtypeStruct((grid * _LB, 128), jnp.int32),
        compiler_params=pltpu.CompilerParams(
            dimension_semantics=("parallel",)),
    )(table_t, table_t, table_t, table_t, eye)


@functools.lru_cache(maxsize=None)
def _make_sc_gather(B, NT):
    bpw = B // _NW
    assert B % (8 * _NW) == 0 and bpw % _CH == 0
    nch = bpw // _CH
    mesh = plsc.VectorSubcoreMesh(core_axis_name="c", subcore_axis_name="s")

    @functools.partial(
        pl.kernel,
        mesh=mesh,
        compiler_params=pltpu.CompilerParams(use_tc_tiling_on_sc=False),
        out_type=jax.ShapeDtypeStruct((B, 32), jnp.int32),
        scratch_types=[
            pltpu.VMEM((bpw,), jnp.int32),
            pltpu.VMEM((bpw, 32), jnp.int32),
            pltpu.SemaphoreType.DMA,
        ],
    )
    def gather_kernel(t_hbm, idx_hbm, out_hbm, idx_v, rows_v, sem):
        wid = lax.axis_index("s") * _NC + lax.axis_index("c")
        base = wid * bpw
        pltpu.sync_copy(idx_hbm.at[pl.ds(base, bpw)], idx_v)
        copies = []
        for t in range(nch):
            sl = pl.ds(t * _CH, _CH)
            copies.append(pltpu.async_copy(t_hbm.at[idx_v.at[sl]],
                                           rows_v.at[sl], sem))
        for c in copies:
            c.wait()
        pltpu.sync_copy(rows_v, out_hbm.at[pl.ds(base, bpw)])

    return gather_kernel


def _unpack32(q32):
    lo = lax.bitcast_convert_type(lax.shift_left(q32, 16), jnp.float32)
    hi = lax.bitcast_convert_type(q32 & jnp.int32(-65536), jnp.float32)
    return lo, hi


def _mlp_body(u_ref, j_ref, w1u_ref, w1j_ref, b1_ref,
              w2_ref, b2_ref, w3_ref, b3_ref, w4_ref, b4_ref,
              o0_ref, o1_ref, o2_ref, o3_ref):
    outs = (o0_ref, o1_ref, o2_ref, o3_ref)
    for q in range(4):
        ulo, uhi = _unpack32(u_ref[:, 32 * q:32 * q + 32])
        jlo, jhi = _unpack32(j_ref[:, 32 * q:32 * q + 32])
        x = jnp.dot(ulo, w1u_ref[:32], preferred_element_type=jnp.float32)
        x = x + jnp.dot(uhi, w1u_ref[32:], preferred_element_type=jnp.float32)
        x = x + jnp.dot(jlo, w1j_ref[:32], preferred_element_type=jnp.float32)
        x = x + jnp.dot(jhi, w1j_ref[32:], preferred_element_type=jnp.float32)
        h = jnp.maximum(x + b1_ref[...], 0.0)
        h = jnp.maximum(
            jnp.dot(h, w2_ref[...],
                    preferred_element_type=jnp.float32) + b2_ref[...], 0.0)
        h = jnp.maximum(
            jnp.dot(h, w3_ref[...],
                    preferred_element_type=jnp.float32) + b3_ref[...], 0.0)
        outs[q][...] = (jnp.sum(h * w4_ref[...], axis=1, keepdims=True)
                        + b4_ref[0, 0])


def _mlp(u32, j32, W1, b1, W2, b2, W3, b3, W4, b4):
    B = u32.shape[0]
    B4 = B // 4
    H1 = W1.shape[0]
    H2 = W2.shape[0]
    H3 = W3.shape[0]
    R = 512
    w1u = W1[:, :64].T
    w1j = W1[:, 64:].T
    full = lambda shape: pl.BlockSpec(shape, lambda i: (0, 0))
    o0, o1, o2, o3 = pl.pallas_call(
        _mlp_body,
        grid=(B4 // R,),
        in_specs=[
            pl.BlockSpec((R, 128), lambda i: (i, 0)),
            pl.BlockSpec((R, 128), lambda i: (i, 0)),
            full((64, H1)),
            full((64, H1)),
            full((1, H1)),
            full((H1, H2)),
            full((1, H2)),
            full((H2, H3)),
            full((1, H3)),
            full((1, H3)),
            full((1, 1)),
        ],
        out_specs=[pl.BlockSpec((R, 1), lambda i: (i, 0))] * 4,
        out_shape=[jax.ShapeDtypeStruct((B4, 1), jnp.float32)] * 4,
        compiler_params=pltpu.CompilerParams(
            dimension_semantics=("parallel",)),
    )(u32.reshape(B4, 128), j32.reshape(B4, 128), w1u, w1j,
      b1.reshape(1, H1), W2.T, b2.reshape(1, H2), W3.T, b3.reshape(1, H3),
      W4.reshape(1, H3), b4.reshape(1, 1))
    return jnp.concatenate([o0, o1, o2, o3], axis=1).reshape(B, 1)


def kernel(user_id, joke_id, user_table, joke_table,
           W1, b1, W2, b2, W3, b3, W4, b4):
    B = user_id.shape[0]
    NU = user_table.shape[0]
    NJ = joke_table.shape[0]
    eye = jnp.eye(64, dtype=jnp.float32)
    to32 = lambda r: ((((r >> (_LG + 2)) << _LG) | (r & (_LB - 1))) << 2) \
        | ((r >> _LG) & 3)
    uid_packed = to32(user_id)
    jid_packed = to32(joke_id)
    jt_packed = _pack(joke_table.T, NJ, eye)
    j32 = _make_sc_gather(B, 4 * jt_packed.shape[0])(
        jt_packed.reshape(-1, 32), jid_packed)
    ut_packed = _pack(user_table.T, NU, eye)
    u32 = _make_sc_gather(B, 4 * ut_packed.shape[0])(
        ut_packed.reshape(-1, 32), uid_packed)
    return _mlp(u32, j32, W1, b1, W2, b2, W3, b3, W4, b4)


# speedup vs baseline: 1.5058x; 1.5058x over previous
import functools

import jax
import jax.numpy as jnp
from jax import lax
from jax.experimental import pallas as pl
from jax.experimental.pallas import tpu as pltpu
from jax.experimental.pallas import tpu_sc as plsc

_NC = 2
_NS = 16
_NW = _NC * _NS
_CH = 128
_LG = 13
_LB = 1 << _LG


def _bf16_bits(y):
    u = lax.bitcast_convert_type(y, jnp.int32)
    t = u + 32767 + (lax.shift_right_logical(u, 16) & 1)
    return lax.shift_right_logical(t, 16)


def _pack_body(x0_ref, x1_ref, x2_ref, x3_ref, eye_ref, o_ref):
    eye = eye_ref[...]
    dn = (((0,), (0,)), ((), ()))
    bits = []
    for ref in (x0_ref, x1_ref, x2_ref, x3_ref):
        y = lax.dot_general(ref[...], eye, dn,
                            preferred_element_type=jnp.float32)
        bits.append(_bf16_bits(y))
    o_ref[:, 0:64] = bits[0] | lax.shift_left(bits[1], 16)
    o_ref[:, 64:128] = bits[2] | lax.shift_left(bits[3], 16)


def _pack(table_t, n_rows, eye):
    nvalid = (n_rows + _LB - 1) // _LB
    grid = (nvalid + 3) // 4
    last = nvalid - 1
    mk = lambda k: pl.BlockSpec(
        (64, _LB), lambda i, k=k: (0, jnp.minimum(4 * i + k, last)))
    return pl.pallas_call(
        _pack_body,
        grid=(grid,),
        in_specs=[mk(0), mk(1), mk(2), mk(3),
                  pl.BlockSpec((64, 64), lambda i: (0, 0))],
        out_specs=pl.BlockSpec((_LB, 128), lambda i: (i, 0)),
        out_shape=jax.ShapeDtypeStruct((grid * _LB, 128), jnp.int32),
        compiler_params=pltpu.CompilerParams(
            dimension_semantics=("parallel",)),
    )(table_t, table_t, table_t, table_t, eye)


@functools.lru_cache(maxsize=None)
def _make_sc_gather(B, NT):
    bpw = B // _NW
    assert B % (8 * _NW) == 0 and bpw % _CH == 0
    nch = bpw // _CH
    mesh = plsc.VectorSubcoreMesh(core_axis_name="c", subcore_axis_name="s")

    @functools.partial(
        pl.kernel,
        mesh=mesh,
        compiler_params=pltpu.CompilerParams(use_tc_tiling_on_sc=False),
        out_type=jax.ShapeDtypeStruct((B, 128), jnp.int32),
        scratch_types=[
            pltpu.VMEM((bpw,), jnp.int32),
            pltpu.VMEM((bpw, 128), jnp.int32),
            pltpu.SemaphoreType.DMA,
        ],
    )
    def gather_kernel(t_hbm, idx_hbm, out_hbm, idx_v, rows_v, sem):
        wid = lax.axis_index("s") * _NC + lax.axis_index("c")
        base = wid * bpw
        pltpu.sync_copy(idx_hbm.at[pl.ds(base, bpw)], idx_v)
        copies = []
        for t in range(nch):
            sl = pl.ds(t * _CH, _CH)
            copies.append(pltpu.async_copy(t_hbm.at[idx_v.at[sl]],
                                           rows_v.at[sl], sem))
        for c in copies:
            c.wait()
        pltpu.sync_copy(rows_v, out_hbm.at[pl.ds(base, bpw)])

    return gather_kernel


def _unpack(words, id_ref):
    hb = jnp.int32(2 * _LB)
    lb = jnp.int32(_LB)
    half = jnp.where((id_ref[...] & hb) == hb, words[:, 64:], words[:, :64])
    bits = jnp.where((id_ref[...] & lb) == lb,
                     half & jnp.int32(-65536), lax.shift_left(half, 16))
    return lax.bitcast_convert_type(bits, jnp.float32)


def _mlp_body(uid_ref, jid_ref, u_ref, j_ref, w1u_ref, w1j_ref, b1_ref,
              w2_ref, b2_ref, w3_ref, b3_ref, w4_ref, b4_ref, o_ref):
    u = _unpack(u_ref[...], uid_ref)
    j = _unpack(j_ref[...], jid_ref)
    x = jnp.dot(u, w1u_ref[...], preferred_element_type=jnp.float32)
    x = x + jnp.dot(j, w1j_ref[...], preferred_element_type=jnp.float32)
    h = jnp.maximum(x + b1_ref[...], 0.0)
    h = jnp.maximum(jnp.dot(h, w2_ref[...],
                            preferred_element_type=jnp.float32) + b2_ref[...],
                    0.0)
    h = jnp.maximum(jnp.dot(h, w3_ref[...],
                            preferred_element_type=jnp.float32) + b3_ref[...],
                    0.0)
    o_ref[...] = (jnp.sum(h * w4_ref[...], axis=1, keepdims=True)
                  + b4_ref[0, 0])


def _mlp(uid, jid, u, j, W1, b1, W2, b2, W3, b3, W4, b4):
    B = u.shape[0]
    H1 = W1.shape[0]
    H2 = W2.shape[0]
    H3 = W3.shape[0]
    R = 2048
    w1u = W1[:, :64].T
    w1j = W1[:, 64:].T
    full = lambda shape: pl.BlockSpec(shape, lambda i: (0, 0))
    return pl.pallas_call(
        _mlp_body,
        grid=(B // R,),
        in_specs=[
            pl.BlockSpec((R, 1), lambda i: (i, 0)),
            pl.BlockSpec((R, 1), lambda i: (i, 0)),
            pl.BlockSpec((R, 128), lambda i: (i, 0)),
            pl.BlockSpec((R, 128), lambda i: (i, 0)),
            full((64, H1)),
            full((64, H1)),
            full((1, H1)),
            full((H1, H2)),
            full((1, H2)),
            full((H2, H3)),
            full((1, H3)),
            full((1, H3)),
            full((1, 1)),
        ],
        out_specs=pl.BlockSpec((R, 1), lambda i: (i, 0)),
        out_shape=jax.ShapeDtypeStruct((B, 1), jnp.float32),
        compiler_params=pltpu.CompilerParams(
            dimension_semantics=("parallel",)),
    )(uid.reshape(B, 1), jid.reshape(B, 1), u, j, w1u, w1j,
      b1.reshape(1, H1), W2.T, b2.reshape(1, H2), W3.T, b3.reshape(1, H3),
      W4.reshape(1, H3), b4.reshape(1, 1))


def kernel(user_id, joke_id, user_table, joke_table,
           W1, b1, W2, b2, W3, b3, W4, b4):
    B = user_id.shape[0]
    NU = user_table.shape[0]
    NJ = joke_table.shape[0]
    eye = jnp.eye(64, dtype=jnp.float32)
    uid_packed = ((user_id >> (_LG + 2)) << _LG) | (user_id & (_LB - 1))
    jid_packed = ((joke_id >> (_LG + 2)) << _LG) | (joke_id & (_LB - 1))
    jt_packed = _pack(joke_table.T, NJ, eye)
    j_rows = _make_sc_gather(B, jt_packed.shape[0])(jt_packed, jid_packed)
    ut_packed = _pack(user_table.T, NU, eye)
    u_rows = _make_sc_gather(B, ut_packed.shape[0])(ut_packed, uid_packed)
    return _mlp(user_id, joke_id, u_rows, j_rows,
                W1, b1, W2, b2, W3, b3, W4, b4)
